# baseline (speedup 1.0000x reference)
import functools

import jax
import jax.numpy as jnp
from jax import lax
from jax.experimental import pallas as pl
from jax.experimental.pallas import tpu as pltpu
from jax.experimental.pallas import tpu_sc as plsc

B = 1024
I = 128
H = 128
L = 2
S = 128


def _make_sc_gather():
    info = plsc.get_sparse_core_info()
    nc, ns = info.num_cores, info.num_subcores
    nw = nc * ns
    bpw = B // nw
    mesh = plsc.VectorSubcoreMesh(core_axis_name="c", subcore_axis_name="s")
    f32 = jnp.float32

    @functools.partial(
        pl.kernel,
        mesh=mesh,
        out_type=[jax.ShapeDtypeStruct((B, H), f32) for _ in range(4)]
        + [jax.ShapeDtypeStruct((B,), jnp.int32)],
        scratch_types=[
            pltpu.VMEM((bpw,), jnp.int32),
            pltpu.VMEM((bpw,), jnp.int32),
            pltpu.VMEM((bpw,), jnp.int32),
            pltpu.VMEM((bpw,), jnp.int32),
            pltpu.VMEM((bpw,), jnp.int32),
            pltpu.VMEM((bpw, H), f32),
            pltpu.VMEM((bpw, H), f32),
            pltpu.VMEM((bpw, H), f32),
            pltpu.VMEM((bpw, H), f32),
            pltpu.SemaphoreType.DMA,
        ],
    )
    def gather(h_tab, c_tab, pos_hbm, op_hbm, gh0, gh1, gc0, gc1, npos,
               pos_v, op_v, idx0_v, idx1_v, npos_v, rh0, rh1, rc0, rc1, sem):
        wid = lax.axis_index("s") * nc + lax.axis_index("c")
        base = wid * bpw
        pltpu.sync_copy(pos_hbm.at[pl.ds(base, bpw)], pos_v)
        pltpu.sync_copy(op_hbm.at[pl.ds(base, bpw)], op_v)
        for j in range(bpw // 16):
            p = pos_v[pl.ds(j * 16, 16)]
            b = base + j * 16 + lax.iota(jnp.int32, 16)
            r2 = (p * B + b) * 2
            idx0_v[pl.ds(j * 16, 16)] = r2
            idx1_v[pl.ds(j * 16, 16)] = r2 + 1
            npos_v[pl.ds(j * 16, 16)] = p + op_v[pl.ds(j * 16, 16)]
        cps = [
            pltpu.async_copy(h_tab.at[idx0_v], rh0, sem),
            pltpu.async_copy(h_tab.at[idx1_v], rh1, sem),
            pltpu.async_copy(c_tab.at[idx0_v], rc0, sem),
            pltpu.async_copy(c_tab.at[idx1_v], rc1, sem),
        ]
        for cp in cps:
            cp.wait()
        pltpu.sync_copy(rh0, gh0.at[pl.ds(base, bpw)])
        pltpu.sync_copy(rh1, gh1.at[pl.ds(base, bpw)])
        pltpu.sync_copy(rc0, gc0.at[pl.ds(base, bpw)])
        pltpu.sync_copy(rc1, gc1.at[pl.ds(base, bpw)])
        pltpu.sync_copy(npos_v, npos.at[pl.ds(base, bpw)])

    return gather


def _lstm_body(x_ref, h0_ref, c0_ref, h1_ref, c1_ref,
               wi0_ref, wh0_ref, b0_ref, wi1_ref, wh1_ref, b1_ref,
               nh0_ref, nc0_ref, nh1_ref, nc1_ref):
    def cell(x, h, c, wi, wh, b):
        dn = (((1,), (1,)), ((), ()))
        g = (lax.dot_general(x, wi, dn, preferred_element_type=jnp.float32)
             + lax.dot_general(h, wh, dn, preferred_element_type=jnp.float32)
             + b)
        i = jax.nn.sigmoid(g[:, 0:H])
        f = jax.nn.sigmoid(g[:, H:2 * H])
        gg = jnp.tanh(g[:, 2 * H:3 * H])
        o = jax.nn.sigmoid(g[:, 3 * H:4 * H])
        c2 = f * c + i * gg
        return o * jnp.tanh(c2), c2

    h0n, c0n = cell(x_ref[...], h0_ref[...], c0_ref[...],
                    wi0_ref[...], wh0_ref[...], b0_ref[...])
    h1n, c1n = cell(h0n, h1_ref[...], c1_ref[...],
                    wi1_ref[...], wh1_ref[...], b1_ref[...])
    nh0_ref[...] = h0n
    nc0_ref[...] = c0n
    nh1_ref[...] = h1n
    nc1_ref[...] = c1n


def _lstm_call(x, h0, c0, h1, c1, wi0, wh0, b0, wi1, wh1, b1):
    f32 = jnp.float32
    return pl.pallas_call(
        _lstm_body,
        out_shape=[
            jax.ShapeDtypeStruct((B, H), f32),
            jax.ShapeDtypeStruct((B, H), f32),
            jax.ShapeDtypeStruct((B, H), f32),
            jax.ShapeDtypeStruct((B, H), f32),
        ],
    )(x, h0, c0, h1, c1, wi0, wh0, b0, wi1, wh1, b1)


SF = 6


def _copy_body(h_ref, c_ref, nh_ref, nc_ref, pos_ref, oh_ref, oc_ref):
    s0 = pl.program_id(0) * SF
    for k in range(SF):
        m = pos_ref[...] == (s0 + k - 1)
        oh_ref[k] = jnp.where(m, nh_ref[...], h_ref[k])
        oc_ref[k] = jnp.where(m, nc_ref[...], c_ref[k])


def _copy_call(h_slab, c_slab, next_h, next_c, pos_rep):
    f32 = jnp.float32
    slab = pl.BlockSpec((SF, B * L, H), lambda s: (s, 0, 0))
    whole = pl.BlockSpec((B * L, H), lambda s: (0, 0))
    return pl.pallas_call(
        _copy_body,
        grid=((S + 1 + SF - 1) // SF,),
        in_specs=[slab, slab, whole, whole,
                  pl.BlockSpec((B * L, 1), lambda s: (0, 0))],
        out_specs=[slab, slab],
        out_shape=[
            jax.ShapeDtypeStruct((S + 1, B * L, H), f32),
            jax.ShapeDtypeStruct((S + 1, B * L, H), f32),
        ],
        compiler_params=pltpu.CompilerParams(
            dimension_semantics=("arbitrary",),
            allow_input_fusion=[False, False, True, True, False],
        ),
    )(h_slab, c_slab, next_h, next_c, pos_rep)


def kernel(input, op, pos, hidden_stack, cell_stack,
           W_ih0, W_hh0, b_ih0, b_hh0, W_ih1, W_hh1, b_ih1, b_hh1):
    h_lh = hidden_stack.transpose(0, 1, 3, 2)
    c_lh = cell_stack.transpose(0, 1, 3, 2)
    h_tab = h_lh.reshape((S + 1) * B * L, H)
    c_tab = c_lh.reshape((S + 1) * B * L, H)
    h_slab = h_lh.reshape(S + 1, B * L, H)
    c_slab = c_lh.reshape(S + 1, B * L, H)

    gh0, gh1, gc0, gc1, new_pos = _make_sc_gather()(h_tab, c_tab, pos, op)

    b0 = (b_ih0 + b_hh0).reshape(1, 4 * H)
    b1 = (b_ih1 + b_hh1).reshape(1, 4 * H)
    nh0, nc0, nh1, nc1 = _lstm_call(
        input, gh0, gc0, gh1, gc1,
        W_ih0, W_hh0, b0, W_ih1, W_hh1, b1)

    next_h = jnp.stack([nh0, nh1], axis=1).reshape(B * L, H)
    next_c = jnp.stack([nc0, nc1], axis=1).reshape(B * L, H)
    pos_rep = jnp.repeat(pos, L).reshape(B * L, 1)

    oh, oc = _copy_call(h_slab, c_slab, next_h, next_c, pos_rep)

    return (oh.reshape(S + 1, B, L, H).transpose(0, 1, 3, 2),
            oc.reshape(S + 1, B, L, H).transpose(0, 1, 3, 2),
            new_pos)


# speedup vs baseline: 1.0078x; 1.0078x over previous
import functools

import jax
import jax.numpy as jnp
from jax import lax
from jax.experimental import pallas as pl
from jax.experimental.pallas import tpu as pltpu
from jax.experimental.pallas import tpu_sc as plsc

B = 1024
I = 128
H = 128
L = 2
S = 128


def _make_sc_gather():
    info = plsc.get_sparse_core_info()
    nc, ns = info.num_cores, info.num_subcores
    nw = nc * ns
    bpw = B // nw
    mesh = plsc.VectorSubcoreMesh(core_axis_name="c", subcore_axis_name="s")
    f32 = jnp.float32

    @functools.partial(
        pl.kernel,
        mesh=mesh,
        out_type=[jax.ShapeDtypeStruct((B, H), f32) for _ in range(4)]
        + [jax.ShapeDtypeStruct((B,), jnp.int32)],
        scratch_types=[
            pltpu.VMEM((bpw,), jnp.int32),
            pltpu.VMEM((bpw,), jnp.int32),
            pltpu.VMEM((bpw,), jnp.int32),
            pltpu.VMEM((bpw,), jnp.int32),
            pltpu.VMEM((bpw,), jnp.int32),
            pltpu.VMEM((bpw, H), f32),
            pltpu.VMEM((bpw, H), f32),
            pltpu.VMEM((bpw, H), f32),
            pltpu.VMEM((bpw, H), f32),
            pltpu.SemaphoreType.DMA,
        ],
    )
    def gather(h_tab, c_tab, pos_hbm, op_hbm, gh0, gh1, gc0, gc1, npos,
               pos_v, op_v, idx0_v, idx1_v, npos_v, rh0, rh1, rc0, rc1, sem):
        wid = lax.axis_index("s") * nc + lax.axis_index("c")
        base = wid * bpw
        pltpu.sync_copy(pos_hbm.at[pl.ds(base, bpw)], pos_v)
        pltpu.sync_copy(op_hbm.at[pl.ds(base, bpw)], op_v)
        for j in range(bpw // 16):
            p = pos_v[pl.ds(j * 16, 16)]
            b = base + j * 16 + lax.iota(jnp.int32, 16)
            r2 = (p * B + b) * 2
            idx0_v[pl.ds(j * 16, 16)] = r2
            idx1_v[pl.ds(j * 16, 16)] = r2 + 1
            npos_v[pl.ds(j * 16, 16)] = p + op_v[pl.ds(j * 16, 16)]
        cps = [
            pltpu.async_copy(h_tab.at[idx0_v], rh0, sem),
            pltpu.async_copy(h_tab.at[idx1_v], rh1, sem),
            pltpu.async_copy(c_tab.at[idx0_v], rc0, sem),
            pltpu.async_copy(c_tab.at[idx1_v], rc1, sem),
        ]
        for cp in cps:
            cp.wait()
        pltpu.sync_copy(rh0, gh0.at[pl.ds(base, bpw)])
        pltpu.sync_copy(rh1, gh1.at[pl.ds(base, bpw)])
        pltpu.sync_copy(rc0, gc0.at[pl.ds(base, bpw)])
        pltpu.sync_copy(rc1, gc1.at[pl.ds(base, bpw)])
        pltpu.sync_copy(npos_v, npos.at[pl.ds(base, bpw)])

    return gather


def _lstm_body(x_ref, h0_ref, c0_ref, h1_ref, c1_ref,
               wi0_ref, wh0_ref, b0_ref, wi1_ref, wh1_ref, b1_ref,
               nh0_ref, nc0_ref, nh1_ref, nc1_ref):
    def cell(x, h, c, wi, wh, b):
        dn = (((1,), (1,)), ((), ()))
        g = (lax.dot_general(x, wi, dn, preferred_element_type=jnp.float32)
             + lax.dot_general(h, wh, dn, preferred_element_type=jnp.float32)
             + b)
        i = jax.nn.sigmoid(g[:, 0:H])
        f = jax.nn.sigmoid(g[:, H:2 * H])
        gg = jnp.tanh(g[:, 2 * H:3 * H])
        o = jax.nn.sigmoid(g[:, 3 * H:4 * H])
        c2 = f * c + i * gg
        return o * jnp.tanh(c2), c2

    h0n, c0n = cell(x_ref[...], h0_ref[...], c0_ref[...],
                    wi0_ref[...], wh0_ref[...], b0_ref[...])
    h1n, c1n = cell(h0n, h1_ref[...], c1_ref[...],
                    wi1_ref[...], wh1_ref[...], b1_ref[...])
    nh0_ref[...] = h0n
    nc0_ref[...] = c0n
    nh1_ref[...] = h1n
    nc1_ref[...] = c1n


def _lstm_call(x, h0, c0, h1, c1, wi0, wh0, b0, wi1, wh1, b1):
    f32 = jnp.float32
    return pl.pallas_call(
        _lstm_body,
        out_shape=[
            jax.ShapeDtypeStruct((B, H), f32),
            jax.ShapeDtypeStruct((B, H), f32),
            jax.ShapeDtypeStruct((B, H), f32),
            jax.ShapeDtypeStruct((B, H), f32),
        ],
    )(x, h0, c0, h1, c1, wi0, wh0, b0, wi1, wh1, b1)


SF = 6


def _copy_body(h_ref, c_ref, nh_ref, nc_ref, pos_ref, oh_ref, oc_ref):
    s0 = pl.program_id(0) * SF
    for k in range(SF):
        m = pos_ref[...] == (s0 + k - 1)
        oh_ref[k] = jnp.where(m, nh_ref[...], h_ref[k])
        oc_ref[k] = jnp.where(m, nc_ref[...], c_ref[k])


def _copy_call(h_slab, c_slab, next_h, next_c, pos_rep):
    f32 = jnp.float32
    slab = pl.BlockSpec((SF, B * L, H), lambda s: (s, 0, 0))
    whole = pl.BlockSpec((B * L, H), lambda s: (0, 0))
    return pl.pallas_call(
        _copy_body,
        grid=((S + 1 + SF - 1) // SF,),
        in_specs=[slab, slab, whole, whole,
                  pl.BlockSpec((B * L, 1), lambda s: (0, 0))],
        out_specs=[slab, slab],
        out_shape=[
            jax.ShapeDtypeStruct((S + 1, B * L, H), f32),
            jax.ShapeDtypeStruct((S + 1, B * L, H), f32),
        ],
        compiler_params=pltpu.CompilerParams(
            dimension_semantics=("arbitrary",),
        ),
    )(h_slab, c_slab, next_h, next_c, pos_rep)


def kernel(input, op, pos, hidden_stack, cell_stack,
           W_ih0, W_hh0, b_ih0, b_hh0, W_ih1, W_hh1, b_ih1, b_hh1):
    h_lh = hidden_stack.transpose(0, 1, 3, 2)
    c_lh = cell_stack.transpose(0, 1, 3, 2)
    h_tab = h_lh.reshape((S + 1) * B * L, H)
    c_tab = c_lh.reshape((S + 1) * B * L, H)
    h_slab = h_lh.reshape(S + 1, B * L, H)
    c_slab = c_lh.reshape(S + 1, B * L, H)

    gh0, gh1, gc0, gc1, new_pos = _make_sc_gather()(h_tab, c_tab, pos, op)

    b0 = (b_ih0 + b_hh0).reshape(1, 4 * H)
    b1 = (b_ih1 + b_hh1).reshape(1, 4 * H)
    nh0, nc0, nh1, nc1 = _lstm_call(
        input, gh0, gc0, gh1, gc1,
        W_ih0, W_hh0, b0, W_ih1, W_hh1, b1)

    next_h = jnp.stack([nh0, nh1], axis=1).reshape(B * L, H)
    next_c = jnp.stack([nc0, nc1], axis=1).reshape(B * L, H)
    pos_rep = jnp.repeat(pos, L).reshape(B * L, 1)

    oh, oc = _copy_call(h_slab, c_slab, next_h, next_c, pos_rep)

    return (oh.reshape(S + 1, B, L, H).transpose(0, 1, 3, 2),
            oc.reshape(S + 1, B, L, H).transpose(0, 1, 3, 2),
            new_pos)
